# baseline (speedup 1.0000x reference)
import jax
import jax.numpy as jnp
from jax import lax
from jax.experimental import pallas as pl
from jax.experimental.pallas import tpu as pltpu
from jax.experimental.pallas import tpu_sc as plsc

B, L, H = 4, 8192, 1024
NC, NS = 2, 16
NW = NC * NS
LPW = L // NW
C = 16
NJ = LPW // C
NM = NJ * B
HV = H // 16
XSLOTS = 4


def _sc_body(x_hbm, emb_hbm, o_hbm, ebuf, xb,
             sx0, sx1, sx2, sx3, so0, so1, so2, so3, se0, se1):
    w = lax.axis_index("s") * NC + lax.axis_index("c")
    l_tile = w * LPW
    sx = (sx0, sx1, sx2, sx3)
    so = (so0, so1, so2, so3)
    se = (se0, se1)

    def x_desc(m, s):
        j = lax.div(m, B)
        b = lax.rem(m, B)
        l0 = l_tile + j * C
        return pltpu.make_async_copy(x_hbm.at[b, pl.ds(l0, C)], xb.at[s], sx[s])

    def o_desc(m, s):
        j = lax.div(m, B)
        b = lax.rem(m, B)
        l0 = l_tile + j * C
        return pltpu.make_async_copy(xb.at[s], o_hbm.at[b, pl.ds(l0, C)], so[s])

    def e_desc(j, q):
        l0 = l_tile + j * C
        return pltpu.make_async_copy(emb_hbm.at[pl.ds(l0, C)], ebuf.at[q], se[q])

    def compute(q, s):
        def row(r, rc):
            for k in range(HV):
                e = ebuf[q, r, pl.ds(k * 16, 16)]
                plsc.addupdate(xb.at[s, r, pl.ds(k * 16, 16)], e)
            return rc

        lax.fori_loop(0, C, row, 0)

    e_desc(0, 0).start()
    e_desc(1, 1).start()
    x_desc(0, 0).start()
    x_desc(1, 1).start()

    def group(j, q):
        e_desc(j, q).wait()
        for b in range(B):
            s = b
            m = B * j + b
            sn = (b + 2) % XSLOTS

            @pl.when(m + 2 < NM)
            def _():
                @pl.when(m >= 2)
                def _():
                    o_desc(m - 2, sn).wait()

                x_desc(m + 2, sn).start()

            x_desc(m, s).wait()
            compute(q, s)
            o_desc(m, s).start()

        @pl.when(j + 2 < NJ)
        def _():
            e_desc(j + 2, q).start()

    def jj_body(jj, carry):
        group(2 * jj, 0)
        group(2 * jj + 1, 1)
        return carry

    lax.fori_loop(0, NJ // 2, jj_body, 0)
    o_desc(NM - 2, 2).wait()
    o_desc(NM - 1, 3).wait()


_run = pl.kernel(
    _sc_body,
    out_type=jax.ShapeDtypeStruct((B, L, H), jnp.float32),
    mesh=plsc.VectorSubcoreMesh(core_axis_name="c", subcore_axis_name="s"),
    scratch_types=[
        pltpu.VMEM((2, C, H), jnp.float32),
        pltpu.VMEM((XSLOTS, C, H), jnp.float32),
    ] + [pltpu.SemaphoreType.DMA] * 10,
)


def kernel(x, pos_emb):
    return _run(x, pos_emb)


# speedup vs baseline: 1.9269x; 1.9269x over previous
import jax
import jax.numpy as jnp
from jax import lax
from jax.experimental import pallas as pl
from jax.experimental.pallas import tpu as pltpu
from jax.experimental.pallas import tpu_sc as plsc

B, L, H = 4, 8192, 1024
NC, NS = 2, 16
NW = NC * NS
LPW = L // NW
C = 8
NCH = LPW // C
HV = H // 16
NSLOT = 3


def _sc_body(x_hbm, emb_hbm, o_hbm, ebuf, xb, si0, si1, si2, so0, so1, so2):
    w = lax.axis_index("s") * NC + lax.axis_index("c")
    l_tile = w * LPW
    si = (si0, si1, si2)
    so = (so0, so1, so2)

    def in_descs(j, p):
        l0 = l_tile + j * C
        return [
            pltpu.make_async_copy(emb_hbm.at[pl.ds(l0, C)], ebuf.at[p], si[p]),
            pltpu.make_async_copy(x_hbm.at[:, pl.ds(l0, C)], xb.at[p], si[p]),
        ]

    def out_descs(j, p):
        l0 = l_tile + j * C
        return [pltpu.make_async_copy(xb.at[p], o_hbm.at[:, pl.ds(l0, C)], so[p])]

    def compute(p):
        def row(r, rc):
            for k in range(HV):
                e = ebuf[p, r, pl.ds(k * 16, 16)]
                for b in range(B):
                    plsc.addupdate(xb.at[p, b, r, pl.ds(k * 16, 16)], e)
            return rc

        lax.fori_loop(0, C, row, 0)

    def step(j, p, pn):
        jn = j + 1

        @pl.when(jn < NCH)
        def _prefetch():
            @pl.when(j >= 2)
            def _drain():
                for d in out_descs(j - 2, pn):
                    d.wait()

            for d in in_descs(jn, pn):
                d.start()

        for d in in_descs(j, p):
            d.wait()
        compute(p)
        for d in out_descs(j, p):
            d.start()

    for d in in_descs(0, 0):
        d.start()
    step(0, 0, 1)
    step(1, 1, 2)

    def g_body(g, carry):
        j0 = 2 + 3 * g
        step(j0, 2, 0)
        step(j0 + 1, 0, 1)
        step(j0 + 2, 1, 2)
        return carry

    lax.fori_loop(0, (NCH - 2) // 3, g_body, 0)

    for d in out_descs(NCH - 2, (NCH - 2) % NSLOT):
        d.wait()
    for d in out_descs(NCH - 1, (NCH - 1) % NSLOT):
        d.wait()


_run = pl.kernel(
    _sc_body,
    out_type=jax.ShapeDtypeStruct((B, L, H), jnp.float32),
    mesh=plsc.VectorSubcoreMesh(core_axis_name="c", subcore_axis_name="s"),
    scratch_types=[
        pltpu.VMEM((NSLOT, C, H), jnp.float32),
        pltpu.VMEM((NSLOT, B, C, H), jnp.float32),
        pltpu.SemaphoreType.DMA,
        pltpu.SemaphoreType.DMA,
        pltpu.SemaphoreType.DMA,
        pltpu.SemaphoreType.DMA,
        pltpu.SemaphoreType.DMA,
        pltpu.SemaphoreType.DMA,
    ],
)


def kernel(x, pos_emb):
    return _run(x, pos_emb)


# speedup vs baseline: 1.9472x; 1.0105x over previous
import jax
import jax.numpy as jnp
from jax import lax
from jax.experimental import pallas as pl
from jax.experimental.pallas import tpu as pltpu
from jax.experimental.pallas import tpu_sc as plsc

B, L, H = 4, 8192, 1024
NC, NS = 2, 16
NW = NC * NS
LPW = L // NW
C = 8
NCH = LPW // C
NG = NCH // 2
HV = H // 16


def _sc_body(x_hbm, emb_hbm, o_hbm, ebuf, xb, si0, si1, so0, so1):
    w = lax.axis_index("s") * NC + lax.axis_index("c")
    l_tile = w * LPW
    si = (si0, si1)
    so = (so0, so1)

    def in_descs(j, p):
        l0 = l_tile + j * C
        d = [pltpu.make_async_copy(emb_hbm.at[pl.ds(l0, C)], ebuf.at[p], si[p])]
        for b in range(B):
            d.append(
                pltpu.make_async_copy(x_hbm.at[b, pl.ds(l0, C)], xb.at[p, b], si[p])
            )
        return d

    def out_descs(j, p):
        l0 = l_tile + j * C
        return [
            pltpu.make_async_copy(xb.at[p, b], o_hbm.at[b, pl.ds(l0, C)], so[p])
            for b in range(B)
        ]

    def compute(p):
        def row(r, rc):
            for k in range(HV):
                e = ebuf[p, r, pl.ds(k * 16, 16)]
                for b in range(B):
                    plsc.addupdate(xb.at[p, b, r, pl.ds(k * 16, 16)], e)
            return rc

        lax.fori_loop(0, C, row, 0)

    for d in in_descs(0, 0):
        d.start()

    def g_body(g, carry):
        for p in (0, 1):
            j = 2 * g + p
            for d in in_descs(j, p):
                d.wait()
            jn = j + 1

            @pl.when(jn < NCH)
            def _start_next():
                @pl.when(jn >= 2)
                def _drain_prev_out():
                    for d in out_descs(jn - 2, 1 - p):
                        d.wait()

                for d in in_descs(jn, 1 - p):
                    d.start()

            compute(p)
            for d in out_descs(j, p):
                d.start()
        return carry

    lax.fori_loop(0, NG, g_body, 0)
    for d in out_descs(NCH - 2, 0):
        d.wait()
    for d in out_descs(NCH - 1, 1):
        d.wait()


_run = pl.kernel(
    _sc_body,
    out_type=jax.ShapeDtypeStruct((B, L, H), jnp.float32),
    mesh=plsc.VectorSubcoreMesh(core_axis_name="c", subcore_axis_name="s"),
    scratch_types=[
        pltpu.VMEM((2, C, H), jnp.float32),
        pltpu.VMEM((2, B, C, H), jnp.float32),
        pltpu.SemaphoreType.DMA,
        pltpu.SemaphoreType.DMA,
        pltpu.SemaphoreType.DMA,
        pltpu.SemaphoreType.DMA,
    ],
)


def kernel(x, pos_emb):
    return _run(x, pos_emb)


# speedup vs baseline: 1.9523x; 1.0027x over previous
import jax
import jax.numpy as jnp
from jax import lax
from jax.experimental import pallas as pl
from jax.experimental.pallas import tpu as pltpu
from jax.experimental.pallas import tpu_sc as plsc

B, L, H = 4, 8192, 1024
NC, NS = 2, 16
NW = NC * NS
LPW = L // NW
C = 8
NCH = LPW // C
NG = NCH // 2
HV = H // 16


def _sc_body(x_hbm, emb_hbm, o_hbm, ebuf, xb, si0, si1, so0, so1):
    w = lax.axis_index("s") * NC + lax.axis_index("c")
    l_tile = w * LPW
    si = (si0, si1)
    so = (so0, so1)

    def in_descs(j, p):
        l0 = l_tile + j * C
        return [
            pltpu.make_async_copy(emb_hbm.at[pl.ds(l0, C)], ebuf.at[p], si[p]),
            pltpu.make_async_copy(x_hbm.at[:, pl.ds(l0, C)], xb.at[p], si[p]),
        ]

    def out_descs(j, p):
        l0 = l_tile + j * C
        return [pltpu.make_async_copy(xb.at[p], o_hbm.at[:, pl.ds(l0, C)], so[p])]

    def compute(p):
        def row(r, rc):
            for k in range(HV):
                e = ebuf[p, r, pl.ds(k * 16, 16)]
                for b in range(B):
                    plsc.addupdate(xb.at[p, b, r, pl.ds(k * 16, 16)], e)
            return rc

        lax.fori_loop(0, C, row, 0)

    for d in in_descs(0, 0):
        d.start()

    def g_body(g, carry):
        for p in (0, 1):
            j = 2 * g + p
            for d in in_descs(j, p):
                d.wait()
            jn = j + 1

            @pl.when(jn < NCH)
            def _start_next():
                @pl.when(jn >= 2)
                def _drain_prev_out():
                    for d in out_descs(jn - 2, 1 - p):
                        d.wait()

                for d in in_descs(jn, 1 - p):
                    d.start()

            compute(p)
            for d in out_descs(j, p):
                d.start()
        return carry

    lax.fori_loop(0, NG, g_body, 0)
    for d in out_descs(NCH - 2, 0):
        d.wait()
    for d in out_descs(NCH - 1, 1):
        d.wait()


_run = pl.kernel(
    _sc_body,
    out_type=jax.ShapeDtypeStruct((B, L, H), jnp.float32),
    mesh=plsc.VectorSubcoreMesh(core_axis_name="c", subcore_axis_name="s"),
    scratch_types=[
        pltpu.VMEM((2, C, H), jnp.float32),
        pltpu.VMEM((2, B, C, H), jnp.float32),
        pltpu.SemaphoreType.DMA,
        pltpu.SemaphoreType.DMA,
        pltpu.SemaphoreType.DMA,
        pltpu.SemaphoreType.DMA,
    ],
)


def kernel(x, pos_emb):
    return _run(x, pos_emb)
